# Initial kernel scaffold; baseline (speedup 1.0000x reference)
import jax
import jax.numpy as jnp
from jax.experimental import pallas as pl


def kernel(user_indices, item_indices, feat0, feat1, feat2, feat3, feat4, feat5, feat6, W_user_mf, W_item_mf, W_user_mlp, W_item_mlp, W_genre, W_sex, W_search, W1, b1, W2, b2, W_out, b_out):
    raise NotImplementedError("write your pallas kernel here")


# speedup vs baseline: 1.2066x; 1.2066x over previous
import functools

import jax
import jax.numpy as jnp
from jax import lax
from jax.experimental import pallas as pl
from jax.experimental.pallas import tpu as pltpu
from jax.experimental.pallas import tpu_sc as plsc

NC, NS = 2, 16
NW = NC * NS
B = 16384
EMB = 64
BPW = B // NW
G = 128
NG = BPW // G
LAYER = 128
BLK = 1024
SMALL = 32


def _sc_gather(w_umlp, w_imlp, w_umf, w_imf, uidx2d, iidx2d):
    mesh = plsc.VectorSubcoreMesh(
        core_axis_name="c", subcore_axis_name="s",
        num_cores=NC, num_subcores=NS)
    row = jax.ShapeDtypeStruct((B, EMB), jnp.float32)

    @functools.partial(
        pl.kernel,
        out_type=(row, row, row, row),
        mesh=mesh,
        scratch_types=[
            pltpu.VMEM((NG, G), jnp.int32),
            pltpu.VMEM((NG, G), jnp.int32),
            pltpu.VMEM((BPW, EMB), jnp.float32),
            pltpu.VMEM((BPW, EMB), jnp.float32),
            pltpu.SemaphoreType.DMA,
        ],
        compiler_params=pltpu.CompilerParams(use_tc_tiling_on_sc=False),
    )
    def k(tu_mlp, ti_mlp, tu_mf, ti_mf, uix_hbm, iix_hbm,
          o_umlp, o_imlp, o_umf, o_imf, uix_v, iix_v, bu, bi, sem):
        wid = lax.axis_index("s") * NC + lax.axis_index("c")
        base = wid * BPW
        pltpu.sync_copy(uix_hbm.at[pl.ds(wid * NG, NG)], uix_v)
        pltpu.sync_copy(iix_hbm.at[pl.ds(wid * NG, NG)], iix_v)
        for tu, ti, ou, oi in ((tu_mlp, ti_mlp, o_umlp, o_imlp),
                               (tu_mf, ti_mf, o_umf, o_imf)):
            waits = []
            for j in range(NG):
                waits.append(pltpu.async_copy(
                    tu.at[uix_v.at[j]], bu.at[pl.ds(j * G, G)], sem))
                waits.append(pltpu.async_copy(
                    ti.at[iix_v.at[j]], bi.at[pl.ds(j * G, G)], sem))
            for w in waits:
                w.wait()
            pltpu.sync_copy(bu, ou.at[pl.ds(base, BPW)])
            pltpu.sync_copy(bi, oi.at[pl.ds(base, BPW)])

    return k(w_umlp, w_imlp, w_umf, w_imf, uidx2d, iidx2d)


def _mlp_body(ru, ri, rumf, rimf, xs, cidx, c_pad, w1s_t, wu_t, wi_t, wf_t,
              b1, w2_t, b2, wo_h, wo_mf, b_out, out_ref):
    f32 = jnp.float32
    h = (jnp.dot(ru[...], wu_t[...], preferred_element_type=f32)
         + jnp.dot(ri[...], wi_t[...], preferred_element_type=f32)
         + jnp.dot(xs[...], wf_t[...], preferred_element_type=f32)
         + b1[...])
    iota = lax.broadcasted_iota(jnp.int32, (BLK, SMALL), 1)
    ci = cidx[...]
    mh = ((iota == ci[:, 0][:, None]).astype(f32)
          + (iota == ci[:, 1][:, None]).astype(f32)
          + (iota == ci[:, 2][:, None]).astype(f32))
    d = jnp.dot(c_pad[...], w1s_t[...], preferred_element_type=f32)
    h = h + jnp.dot(mh, d, preferred_element_type=f32)
    h = jnp.maximum(h, 0.0)
    h2 = jnp.maximum(jnp.dot(h, w2_t[...], preferred_element_type=f32)
                     + b2[...], 0.0)
    mf = rumf[...] * rimf[...]
    out = (jnp.dot(h2, wo_h[...], preferred_element_type=f32)
           + jnp.dot(mf, wo_mf[...], preferred_element_type=f32)
           + b_out[...])
    out_ref[...] = out


def _tc_mlp(ru, ri, rumf, rimf, xs, cidx, c_pad, w1s_t, wu_t, wi_t, wf_t,
            b1, w2_t, b2, wo_h, wo_mf, b_out):
    nblk = B // BLK
    rows = lambda shp: pl.BlockSpec((BLK,) + shp[1:], lambda i: (i,) + (0,) * (len(shp) - 1))
    full = lambda shp: pl.BlockSpec(shp, lambda i: (0,) * len(shp))
    in_specs = [
        rows((B, EMB)), rows((B, EMB)), rows((B, EMB)), rows((B, EMB)),
        rows((B, 8)), rows((B, 4)),
        full(c_pad.shape), full(w1s_t.shape), full(wu_t.shape),
        full(wi_t.shape), full(wf_t.shape), full(b1.shape),
        full(w2_t.shape), full(b2.shape), full(wo_h.shape),
        full(wo_mf.shape), full(b_out.shape),
    ]
    return pl.pallas_call(
        _mlp_body,
        grid=(nblk,),
        in_specs=in_specs,
        out_specs=pl.BlockSpec((BLK, 1), lambda i: (i, 0)),
        out_shape=jax.ShapeDtypeStruct((B, 1), jnp.float32),
    )(ru, ri, rumf, rimf, xs, cidx, c_pad, w1s_t, wu_t, wi_t, wf_t,
      b1, w2_t, b2, wo_h, wo_mf, b_out)


def kernel(user_indices, item_indices, feat0, feat1, feat2, feat3, feat4,
           feat5, feat6, W_user_mf, W_item_mf, W_user_mlp, W_item_mlp,
           W_genre, W_sex, W_search, W1, b1, W2, b2, W_out, b_out):
    uidx = user_indices.astype(jnp.int32).reshape(NW * NG, G)
    iidx = item_indices.astype(jnp.int32).reshape(NW * NG, G)

    ru, ri, rumf, rimf = _sc_gather(
        W_user_mlp, W_item_mlp, W_user_mf, W_item_mf, uidx, iidx)

    xs = jnp.stack([feat0, feat1, feat4, feat6], axis=1)
    xs = jnp.pad(xs, ((0, 0), (0, 4)))
    cidx = jnp.stack([feat3.astype(jnp.int32),
                      feat2.astype(jnp.int32) + 18,
                      feat5.astype(jnp.int32) + 20,
                      jnp.full((B,), -1, jnp.int32)], axis=1)
    c_pad = jnp.zeros((SMALL, 16), jnp.float32)
    c_pad = c_pad.at[0:18, 0:9].set(W_genre)
    c_pad = c_pad.at[18:20, 9:10].set(W_sex)
    c_pad = c_pad.at[20:30, 10:15].set(W_search)
    wu_t = W1[:, 0:64].T
    wi_t = W1[:, 64:128].T
    w1s_t = jnp.pad(W1[:, 128:143].T, ((0, 1), (0, 0)))
    wf_t = jnp.pad(W1[:, 143:147].T, ((0, 4), (0, 0)))
    w2_t = W2.T
    wo_h = W_out[0, 0:64][:, None]
    wo_mf = W_out[0, 64:128][:, None]

    out = _tc_mlp(ru, ri, rumf, rimf, xs, cidx, c_pad, w1s_t, wu_t, wi_t,
                  wf_t, b1[None, :], w2_t, b2[None, :], wo_h, wo_mf,
                  b_out[None, :])
    return out[:, 0]


# speedup vs baseline: 1.4512x; 1.2028x over previous
import functools

import jax
import jax.numpy as jnp
from jax import lax
from jax.experimental import pallas as pl
from jax.experimental.pallas import tpu as pltpu
from jax.experimental.pallas import tpu_sc as plsc

NC, NS = 2, 16
NW = NC * NS
B = 16384
EMB = 64
CAT = 2 * EMB
BPW = B // NW
G = 128
HALF = 256
LAYER = 128
BLK = 1024
SMALL = 32


def _sc_gather(tab_u, tab_i, uidx, iidx):
    mesh = plsc.VectorSubcoreMesh(
        core_axis_name="c", subcore_axis_name="s",
        num_cores=NC, num_subcores=NS)
    out = jax.ShapeDtypeStruct((B, CAT), jnp.float32)

    @functools.partial(
        pl.kernel,
        out_type=(out, out),
        mesh=mesh,
        scratch_types=[
            pltpu.VMEM((BPW,), jnp.int32),
            pltpu.VMEM((BPW,), jnp.int32),
            pltpu.VMEM((HALF, CAT), jnp.float32),
            pltpu.VMEM((HALF, CAT), jnp.float32),
            pltpu.SemaphoreType.DMA,
        ],
        compiler_params=pltpu.CompilerParams(use_tc_tiling_on_sc=True),
    )
    def k(tu, ti, uix_hbm, iix_hbm, ou, oi, uv, iv, bu, bi, sem):
        wid = lax.axis_index("s") * NC + lax.axis_index("c")
        base = wid * BPW
        pltpu.sync_copy(uix_hbm.at[pl.ds(base, BPW)], uv)
        pltpu.sync_copy(iix_hbm.at[pl.ds(base, BPW)], iv)
        for h in range(BPW // HALF):
            waits = []
            for j in range(HALF // G):
                o = h * HALF + j * G
                waits.append(pltpu.async_copy(
                    tu.at[uv.at[pl.ds(o, G)]], bu.at[pl.ds(j * G, G)], sem))
                waits.append(pltpu.async_copy(
                    ti.at[iv.at[pl.ds(o, G)]], bi.at[pl.ds(j * G, G)], sem))
            for w in waits:
                w.wait()
            pltpu.sync_copy(bu, ou.at[pl.ds(base + h * HALF, HALF)])
            pltpu.sync_copy(bi, oi.at[pl.ds(base + h * HALF, HALF)])

    return k(tab_u, tab_i, uidx, iidx)


def _mlp_body(ru, ri, xs, cidx, c_pad, w1s_t, wu_t, wi_t, wf_t,
              b1, w2_t, b2, wo_h, wo_mf, b_out, out_ref):
    f32 = jnp.float32
    u = ru[...]
    i = ri[...]
    h = (jnp.dot(u, wu_t[...], preferred_element_type=f32)
         + jnp.dot(i, wi_t[...], preferred_element_type=f32)
         + jnp.dot(xs[...], wf_t[...], preferred_element_type=f32)
         + b1[...])
    iota = lax.broadcasted_iota(jnp.int32, (BLK, SMALL), 1)
    ci = cidx[...]
    mh = ((iota == ci[:, 0][:, None]).astype(f32)
          + (iota == ci[:, 1][:, None]).astype(f32)
          + (iota == ci[:, 2][:, None]).astype(f32))
    d = jnp.dot(c_pad[...], w1s_t[...], preferred_element_type=f32)
    h = h + jnp.dot(mh, d, preferred_element_type=f32)
    h = jnp.maximum(h, 0.0)
    h2 = jnp.maximum(jnp.dot(h, w2_t[...], preferred_element_type=f32)
                     + b2[...], 0.0)
    out = (jnp.dot(h2, wo_h[...], preferred_element_type=f32)
           + jnp.dot(u * i, wo_mf[...], preferred_element_type=f32)
           + b_out[...])
    out_ref[...] = out


def _tc_mlp(ru, ri, xs, cidx, c_pad, w1s_t, wu_t, wi_t, wf_t,
            b1, w2_t, b2, wo_h, wo_mf, b_out):
    nblk = B // BLK
    rows = lambda shp: pl.BlockSpec((BLK,) + shp[1:], lambda i: (i,) + (0,) * (len(shp) - 1))
    full = lambda shp: pl.BlockSpec(shp, lambda i: (0,) * len(shp))
    in_specs = [
        rows((B, CAT)), rows((B, CAT)), rows((B, 8)), rows((B, 4)),
        full(c_pad.shape), full(w1s_t.shape), full(wu_t.shape),
        full(wi_t.shape), full(wf_t.shape), full(b1.shape),
        full(w2_t.shape), full(b2.shape), full(wo_h.shape),
        full(wo_mf.shape), full(b_out.shape),
    ]
    return pl.pallas_call(
        _mlp_body,
        grid=(nblk,),
        in_specs=in_specs,
        out_specs=pl.BlockSpec((BLK, 1), lambda i: (i, 0)),
        out_shape=jax.ShapeDtypeStruct((B, 1), jnp.float32),
    )(ru, ri, xs, cidx, c_pad, w1s_t, wu_t, wi_t, wf_t,
      b1, w2_t, b2, wo_h, wo_mf, b_out)


def kernel(user_indices, item_indices, feat0, feat1, feat2, feat3, feat4,
           feat5, feat6, W_user_mf, W_item_mf, W_user_mlp, W_item_mlp,
           W_genre, W_sex, W_search, W1, b1, W2, b2, W_out, b_out):
    tab_u = jnp.concatenate([W_user_mlp, W_user_mf], axis=1)
    tab_i = jnp.concatenate([W_item_mlp, W_item_mf], axis=1)
    uidx = user_indices.astype(jnp.int32)
    iidx = item_indices.astype(jnp.int32)

    ru, ri = _sc_gather(tab_u, tab_i, uidx, iidx)

    xs = jnp.stack([feat0, feat1, feat4, feat6], axis=1)
    xs = jnp.pad(xs, ((0, 0), (0, 4)))
    cidx = jnp.stack([feat3.astype(jnp.int32),
                      feat2.astype(jnp.int32) + 18,
                      feat5.astype(jnp.int32) + 20,
                      jnp.full((B,), -1, jnp.int32)], axis=1)
    c_pad = jnp.zeros((SMALL, 16), jnp.float32)
    c_pad = c_pad.at[0:18, 0:9].set(W_genre)
    c_pad = c_pad.at[18:20, 9:10].set(W_sex)
    c_pad = c_pad.at[20:30, 10:15].set(W_search)
    wu_t = jnp.pad(W1[:, 0:64].T, ((0, 64), (0, 0)))
    wi_t = jnp.pad(W1[:, 64:128].T, ((0, 64), (0, 0)))
    w1s_t = jnp.pad(W1[:, 128:143].T, ((0, 1), (0, 0)))
    wf_t = jnp.pad(W1[:, 143:147].T, ((0, 4), (0, 0)))
    w2_t = W2.T
    wo_h = W_out[0, 0:64][:, None]
    wo_mf = jnp.pad(W_out[0, 64:128][:, None], ((64, 0), (0, 0)))

    out = _tc_mlp(ru, ri, xs, cidx, c_pad, w1s_t, wu_t, wi_t,
                  wf_t, b1[None, :], w2_t, b2[None, :], wo_h, wo_mf,
                  b_out[None, :])
    return out[:, 0]


# speedup vs baseline: 2.2577x; 1.5557x over previous
import functools

import jax
import jax.numpy as jnp
from jax import lax
from jax.experimental import pallas as pl
from jax.experimental.pallas import tpu as pltpu
from jax.experimental.pallas import tpu_sc as plsc

NC, NS = 2, 16
NW = NC * NS
B = 16384
EMB = 64
CAT = 2 * EMB
BPW = B // NW
G = 128
HALF = 256
LAYER = 128
BLK = 1024
SMALL = 32


N_ROWS = 100000
TR_C = 4096


def _tr_body(tum, tuf, tim, tif, eye, ou, oi):
    f32 = jnp.float32
    dn = (((0,), (0,)), ((), ()))
    e = eye[...]
    ou[...] = jnp.concatenate(
        [lax.dot_general(tum[...], e, dn, preferred_element_type=f32),
         lax.dot_general(tuf[...], e, dn, preferred_element_type=f32)], axis=1)
    oi[...] = jnp.concatenate(
        [lax.dot_general(tim[...], e, dn, preferred_element_type=f32),
         lax.dot_general(tif[...], e, dn, preferred_element_type=f32)], axis=1)


def _transpose_concat(tt_um, tt_uf, tt_im, tt_if):
    nblk = pl.cdiv(N_ROWS, TR_C)
    col = pl.BlockSpec((EMB, TR_C), lambda i: (0, i))
    out = jax.ShapeDtypeStruct((N_ROWS, CAT), jnp.float32)
    return pl.pallas_call(
        _tr_body,
        grid=(nblk,),
        in_specs=[col, col, col, col,
                  pl.BlockSpec((EMB, EMB), lambda i: (0, 0))],
        out_specs=(pl.BlockSpec((TR_C, CAT), lambda i: (i, 0)),
                   pl.BlockSpec((TR_C, CAT), lambda i: (i, 0))),
        out_shape=(out, out),
    )(tt_um, tt_uf, tt_im, tt_if, jnp.eye(EMB, dtype=jnp.float32))


def _sc_gather(tab_u, tab_i, uidx, iidx):
    mesh = plsc.VectorSubcoreMesh(
        core_axis_name="c", subcore_axis_name="s",
        num_cores=NC, num_subcores=NS)
    out = jax.ShapeDtypeStruct((B, CAT), jnp.float32)

    @functools.partial(
        pl.kernel,
        out_type=(out, out),
        mesh=mesh,
        scratch_types=[
            pltpu.VMEM((BPW,), jnp.int32),
            pltpu.VMEM((BPW,), jnp.int32),
            pltpu.VMEM((HALF, CAT), jnp.float32),
            pltpu.VMEM((HALF, CAT), jnp.float32),
            pltpu.SemaphoreType.DMA,
        ],
        compiler_params=pltpu.CompilerParams(use_tc_tiling_on_sc=True),
    )
    def k(tu, ti, uix_hbm, iix_hbm, ou, oi, uv, iv, bu, bi, sem):
        wid = lax.axis_index("s") * NC + lax.axis_index("c")
        base = wid * BPW
        pltpu.sync_copy(uix_hbm.at[pl.ds(base, BPW)], uv)
        pltpu.sync_copy(iix_hbm.at[pl.ds(base, BPW)], iv)
        for h in range(BPW // HALF):
            waits = []
            for j in range(HALF // G):
                o = h * HALF + j * G
                waits.append(pltpu.async_copy(
                    tu.at[uv.at[pl.ds(o, G)]], bu.at[pl.ds(j * G, G)], sem))
                waits.append(pltpu.async_copy(
                    ti.at[iv.at[pl.ds(o, G)]], bi.at[pl.ds(j * G, G)], sem))
            for w in waits:
                w.wait()
            pltpu.sync_copy(bu, ou.at[pl.ds(base + h * HALF, HALF)])
            pltpu.sync_copy(bi, oi.at[pl.ds(base + h * HALF, HALF)])

    return k(tab_u, tab_i, uidx, iidx)


def _mlp_body(ru, ri, xs, cidx, c_pad, w1s_t, wu_t, wi_t, wf_t,
              b1, w2_t, b2, wo_h, wo_mf, b_out, out_ref):
    f32 = jnp.float32
    u = ru[...]
    i = ri[...]
    h = (jnp.dot(u, wu_t[...], preferred_element_type=f32)
         + jnp.dot(i, wi_t[...], preferred_element_type=f32)
         + jnp.dot(xs[...], wf_t[...], preferred_element_type=f32)
         + b1[...])
    iota = lax.broadcasted_iota(jnp.int32, (BLK, SMALL), 1)
    ci = cidx[...]
    mh = ((iota == ci[:, 0][:, None]).astype(f32)
          + (iota == ci[:, 1][:, None]).astype(f32)
          + (iota == ci[:, 2][:, None]).astype(f32))
    d = jnp.dot(c_pad[...], w1s_t[...], preferred_element_type=f32)
    h = h + jnp.dot(mh, d, preferred_element_type=f32)
    h = jnp.maximum(h, 0.0)
    h2 = jnp.maximum(jnp.dot(h, w2_t[...], preferred_element_type=f32)
                     + b2[...], 0.0)
    out = (jnp.dot(h2, wo_h[...], preferred_element_type=f32)
           + jnp.dot(u * i, wo_mf[...], preferred_element_type=f32)
           + b_out[...])
    out_ref[...] = out


def _tc_mlp(ru, ri, xs, cidx, c_pad, w1s_t, wu_t, wi_t, wf_t,
            b1, w2_t, b2, wo_h, wo_mf, b_out):
    nblk = B // BLK
    rows = lambda shp: pl.BlockSpec((BLK,) + shp[1:], lambda i: (i,) + (0,) * (len(shp) - 1))
    full = lambda shp: pl.BlockSpec(shp, lambda i: (0,) * len(shp))
    in_specs = [
        rows((B, CAT)), rows((B, CAT)), rows((B, 8)), rows((B, 4)),
        full(c_pad.shape), full(w1s_t.shape), full(wu_t.shape),
        full(wi_t.shape), full(wf_t.shape), full(b1.shape),
        full(w2_t.shape), full(b2.shape), full(wo_h.shape),
        full(wo_mf.shape), full(b_out.shape),
    ]
    return pl.pallas_call(
        _mlp_body,
        grid=(nblk,),
        in_specs=in_specs,
        out_specs=pl.BlockSpec((BLK, 1), lambda i: (i, 0)),
        out_shape=jax.ShapeDtypeStruct((B, 1), jnp.float32),
    )(ru, ri, xs, cidx, c_pad, w1s_t, wu_t, wi_t, wf_t,
      b1, w2_t, b2, wo_h, wo_mf, b_out)


def kernel(user_indices, item_indices, feat0, feat1, feat2, feat3, feat4,
           feat5, feat6, W_user_mf, W_item_mf, W_user_mlp, W_item_mlp,
           W_genre, W_sex, W_search, W1, b1, W2, b2, W_out, b_out):
    tab_u, tab_i = _transpose_concat(
        W_user_mlp.T, W_user_mf.T, W_item_mlp.T, W_item_mf.T)
    uidx = user_indices.astype(jnp.int32)
    iidx = item_indices.astype(jnp.int32)

    ru, ri = _sc_gather(tab_u, tab_i, uidx, iidx)

    xs = jnp.stack([feat0, feat1, feat4, feat6], axis=1)
    xs = jnp.pad(xs, ((0, 0), (0, 4)))
    cidx = jnp.stack([feat3.astype(jnp.int32),
                      feat2.astype(jnp.int32) + 18,
                      feat5.astype(jnp.int32) + 20,
                      jnp.full((B,), -1, jnp.int32)], axis=1)
    c_pad = jnp.zeros((SMALL, 16), jnp.float32)
    c_pad = c_pad.at[0:18, 0:9].set(W_genre)
    c_pad = c_pad.at[18:20, 9:10].set(W_sex)
    c_pad = c_pad.at[20:30, 10:15].set(W_search)
    wu_t = jnp.pad(W1[:, 0:64].T, ((0, 64), (0, 0)))
    wi_t = jnp.pad(W1[:, 64:128].T, ((0, 64), (0, 0)))
    w1s_t = jnp.pad(W1[:, 128:143].T, ((0, 1), (0, 0)))
    wf_t = jnp.pad(W1[:, 143:147].T, ((0, 4), (0, 0)))
    w2_t = W2.T
    wo_h = W_out[0, 0:64][:, None]
    wo_mf = jnp.pad(W_out[0, 64:128][:, None], ((64, 0), (0, 0)))

    out = _tc_mlp(ru, ri, xs, cidx, c_pad, w1s_t, wu_t, wi_t,
                  wf_t, b1[None, :], w2_t, b2[None, :], wo_h, wo_mf,
                  b_out[None, :])
    return out[:, 0]
